# baseline (device time: 149648 ns/iter reference)
import jax
import jax.numpy as jnp
from jax import lax
from jax.experimental import pallas as pl
from jax.experimental.pallas import tpu as pltpu

N_DEV = 4


def kernel(A, B):
    m, k = A.shape
    _, n = B.shape

    def body(a_ref, b_ref, out_ref, comm_ref, send_sems, recv_sems):
        my_pos = lax.axis_index("i")
        left = (my_pos - 1) % N_DEV
        right = (my_pos + 1) % N_DEV

        barrier_sem = pltpu.get_barrier_semaphore()
        for nbr in [left, right]:
            pl.semaphore_signal(
                barrier_sem, inc=1,
                device_id=(nbr,), device_id_type=pl.DeviceIdType.MESH,
            )
        pl.semaphore_wait(barrier_sem, 2)

        partial = jnp.dot(a_ref[:, :], b_ref[:, :],
                          preferred_element_type=jnp.float32)
        out_ref[:, :] = partial
        comm_ref[0, :, :] = partial

        for h in range(N_DEV - 1):
            rdma = pltpu.make_async_remote_copy(
                src_ref=comm_ref.at[h],
                dst_ref=comm_ref.at[h + 1],
                send_sem=send_sems.at[h],
                recv_sem=recv_sems.at[h],
                device_id=(right,),
                device_id_type=pl.DeviceIdType.MESH,
            )
            rdma.start()
            rdma.wait()
            out_ref[:, :] = out_ref[:, :] + comm_ref[h + 1, :, :]

    return pl.pallas_call(
        body,
        out_shape=jax.ShapeDtypeStruct((m, n), jnp.float32),
        in_specs=[
            pl.BlockSpec(memory_space=pltpu.VMEM),
            pl.BlockSpec(memory_space=pltpu.VMEM),
        ],
        out_specs=pl.BlockSpec(memory_space=pltpu.VMEM),
        scratch_shapes=[
            pltpu.VMEM((N_DEV, m, n), jnp.float32),
            pltpu.SemaphoreType.DMA((N_DEV - 1,)),
            pltpu.SemaphoreType.DMA((N_DEV - 1,)),
        ],
        compiler_params=pltpu.CompilerParams(collective_id=0),
    )(A, B)


# device time: 53531 ns/iter; 2.7955x vs baseline; 2.7955x over previous
import jax
import jax.numpy as jnp
from jax import lax
from jax.experimental import pallas as pl
from jax.experimental.pallas import tpu as pltpu

N_DEV = 4


def kernel(A, B):
    m, k = A.shape
    _, n = B.shape
    half = m // 2
    rows = half // N_DEV

    def body(a_ref, b_ref, out_ref, rs_buf,
             rs_send, rs_recv, ag_send, ag_recv):
        my = lax.axis_index("i")
        left = (my - 1) % N_DEV
        right = (my + 1) % N_DEV

        def cw_rows(c):
            return pl.ds((c % N_DEV) * rows, rows)

        def ccw_rows(c):
            return pl.ds(half + (c % N_DEV) * rows, rows)

        barrier_sem = pltpu.get_barrier_semaphore()
        for nbr in [left, right]:
            pl.semaphore_signal(
                barrier_sem, inc=1,
                device_id=(nbr,), device_id_type=pl.DeviceIdType.MESH,
            )
        pl.semaphore_wait(barrier_sem, 2)

        out_ref[:, :] = jnp.dot(a_ref[:, :], b_ref[:, :],
                                preferred_element_type=jnp.float32)

        for s in range(N_DEV - 1):
            send_cw = pltpu.make_async_remote_copy(
                src_ref=out_ref.at[cw_rows(my - s), :],
                dst_ref=rs_buf.at[0, s],
                send_sem=rs_send.at[0, s],
                recv_sem=rs_recv.at[0, s],
                device_id=(right,),
                device_id_type=pl.DeviceIdType.MESH,
            )
            send_ccw = pltpu.make_async_remote_copy(
                src_ref=out_ref.at[ccw_rows(my + s), :],
                dst_ref=rs_buf.at[1, s],
                send_sem=rs_send.at[1, s],
                recv_sem=rs_recv.at[1, s],
                device_id=(left,),
                device_id_type=pl.DeviceIdType.MESH,
            )
            send_cw.start()
            send_ccw.start()
            send_cw.wait()
            send_ccw.wait()
            out_ref[cw_rows(my - 1 - s), :] = (
                out_ref[cw_rows(my - 1 - s), :] + rs_buf[0, s, :, :]
            )
            out_ref[ccw_rows(my + 1 + s), :] = (
                out_ref[ccw_rows(my + 1 + s), :] + rs_buf[1, s, :, :]
            )


        for t in range(N_DEV - 1):
            send_cw = pltpu.make_async_remote_copy(
                src_ref=out_ref.at[cw_rows(my + 1 - t), :],
                dst_ref=out_ref.at[cw_rows(my + 1 - t), :],
                send_sem=ag_send.at[0, t],
                recv_sem=ag_recv.at[0, t],
                device_id=(right,),
                device_id_type=pl.DeviceIdType.MESH,
            )
            send_ccw = pltpu.make_async_remote_copy(
                src_ref=out_ref.at[ccw_rows(my - 1 + t), :],
                dst_ref=out_ref.at[ccw_rows(my - 1 + t), :],
                send_sem=ag_send.at[1, t],
                recv_sem=ag_recv.at[1, t],
                device_id=(left,),
                device_id_type=pl.DeviceIdType.MESH,
            )
            send_cw.start()
            send_ccw.start()
            send_cw.wait()
            send_ccw.wait()

    return pl.pallas_call(
        body,
        out_shape=jax.ShapeDtypeStruct((m, n), jnp.float32),
        in_specs=[
            pl.BlockSpec(memory_space=pltpu.VMEM),
            pl.BlockSpec(memory_space=pltpu.VMEM),
        ],
        out_specs=pl.BlockSpec(memory_space=pltpu.VMEM),
        scratch_shapes=[
            pltpu.VMEM((2, N_DEV - 1, rows, n), jnp.float32),
            pltpu.SemaphoreType.DMA((2, N_DEV - 1)),
            pltpu.SemaphoreType.DMA((2, N_DEV - 1)),
            pltpu.SemaphoreType.DMA((2, N_DEV - 1)),
            pltpu.SemaphoreType.DMA((2, N_DEV - 1)),
        ],
        compiler_params=pltpu.CompilerParams(collective_id=0),
    )(A, B)


# device time: 49518 ns/iter; 3.0221x vs baseline; 1.0810x over previous
import jax
import jax.numpy as jnp
from jax import lax
from jax.experimental import pallas as pl
from jax.experimental.pallas import tpu as pltpu

N_DEV = 4


def kernel(A, B):
    m, k = A.shape
    _, n = B.shape
    half = m // 2
    sub = m // 4
    qtr = m // 8

    def body(a_ref, b_ref, out_ref, p1_buf, p2_buf, send_sems, recv_sems):
        my = lax.axis_index("i")
        a_bit = my % 2
        b_bit = my // 2
        pa = my + 1 - 2 * a_bit
        pb = 3 - my

        ka = (a_bit + b_bit) % 2
        kb = b_bit
        ka2 = b_bit
        kb2 = a_bit

        send_a = (1 - ka) * sub
        keep_a = ka * sub
        send_b = half + (1 - ka2) * sub
        keep_b = half + ka2 * sub

        qsend_a = keep_a + (1 - kb) * qtr
        qkeep_a = keep_a + kb * qtr
        qsend_b = keep_b + (1 - kb2) * qtr
        qkeep_b = keep_b + kb2 * qtr

        def rc(src, dst, ph, hf, dev):
            return pltpu.make_async_remote_copy(
                src_ref=src, dst_ref=dst,
                send_sem=send_sems.at[ph, hf],
                recv_sem=recv_sems.at[ph, hf],
                device_id=(dev,), device_id_type=pl.DeviceIdType.MESH,
            )

        def mm(rows, nrows):
            return jnp.dot(a_ref[pl.ds(rows, nrows), :], b_ref[:, :],
                           preferred_element_type=jnp.float32)

        barrier_sem = pltpu.get_barrier_semaphore()
        for nbr in [pa, pb]:
            pl.semaphore_signal(
                barrier_sem, inc=1,
                device_id=(nbr,), device_id_type=pl.DeviceIdType.MESH,
            )
        pl.semaphore_wait(barrier_sem, 2)

        out_ref[pl.ds(send_a, sub), :] = mm(send_a, sub)
        out_ref[pl.ds(send_b, sub), :] = mm(send_b, sub)

        p1a = rc(out_ref.at[pl.ds(send_a, sub), :], p1_buf.at[0], 0, 0, pa)
        p1b = rc(out_ref.at[pl.ds(send_b, sub), :], p1_buf.at[1], 0, 1, pb)
        p1a.start()
        p1b.start()

        out_ref[pl.ds(keep_a, sub), :] = mm(keep_a, sub)
        out_ref[pl.ds(keep_b, sub), :] = mm(keep_b, sub)

        p1a.wait_recv()
        p1b.wait_recv()
        out_ref[pl.ds(keep_a, sub), :] = (
            out_ref[pl.ds(keep_a, sub), :] + p1_buf[0, :, :]
        )
        out_ref[pl.ds(keep_b, sub), :] = (
            out_ref[pl.ds(keep_b, sub), :] + p1_buf[1, :, :]
        )

        p2a = rc(out_ref.at[pl.ds(qsend_a, qtr), :], p2_buf.at[0], 1, 0, pb)
        p2b = rc(out_ref.at[pl.ds(qsend_b, qtr), :], p2_buf.at[1], 1, 1, pa)
        p2a.start()
        p2b.start()
        p2a.wait_recv()
        p2b.wait_recv()
        out_ref[pl.ds(qkeep_a, qtr), :] = (
            out_ref[pl.ds(qkeep_a, qtr), :] + p2_buf[0, :, :]
        )
        out_ref[pl.ds(qkeep_b, qtr), :] = (
            out_ref[pl.ds(qkeep_b, qtr), :] + p2_buf[1, :, :]
        )

        p3a = rc(out_ref.at[pl.ds(qkeep_a, qtr), :],
                 out_ref.at[pl.ds(qkeep_a, qtr), :], 2, 0, pb)
        p3b = rc(out_ref.at[pl.ds(qkeep_b, qtr), :],
                 out_ref.at[pl.ds(qkeep_b, qtr), :], 2, 1, pa)
        p3a.start()
        p3b.start()
        p3a.wait_recv()
        p3b.wait_recv()

        p4a = rc(out_ref.at[pl.ds(keep_a, sub), :],
                 out_ref.at[pl.ds(keep_a, sub), :], 3, 0, pa)
        p4b = rc(out_ref.at[pl.ds(keep_b, sub), :],
                 out_ref.at[pl.ds(keep_b, sub), :], 3, 1, pb)
        p4a.start()
        p4b.start()
        p4a.wait_recv()
        p4b.wait_recv()

        for d in [p1a, p1b, p2a, p2b, p3a, p3b, p4a, p4b]:
            d.wait_send()

    return pl.pallas_call(
        body,
        out_shape=jax.ShapeDtypeStruct((m, n), jnp.float32),
        in_specs=[
            pl.BlockSpec(memory_space=pltpu.VMEM),
            pl.BlockSpec(memory_space=pltpu.VMEM),
        ],
        out_specs=pl.BlockSpec(memory_space=pltpu.VMEM),
        scratch_shapes=[
            pltpu.VMEM((2, sub, n), jnp.float32),
            pltpu.VMEM((2, qtr, n), jnp.float32),
            pltpu.SemaphoreType.DMA((4, 2)),
            pltpu.SemaphoreType.DMA((4, 2)),
        ],
        compiler_params=pltpu.CompilerParams(collective_id=0),
    )(A, B)


# device time: 47945 ns/iter; 3.1212x vs baseline; 1.0328x over previous
import jax
import jax.numpy as jnp
from jax import lax
from jax.experimental import pallas as pl
from jax.experimental.pallas import tpu as pltpu

N_DEV = 4


def kernel(A, B):
    m, k = A.shape
    _, n = B.shape
    half = m // 2
    sub = m // 4
    qtr = m // 8

    def body(a_ref, b_ref, out_ref, p1_buf, p2_buf, send_sems, recv_sems):
        my = lax.axis_index("i")
        a_bit = my % 2
        b_bit = my // 2
        pa = my + 1 - 2 * a_bit
        pb = 3 - my

        ka = (a_bit + b_bit) % 2
        kb = b_bit
        ka2 = b_bit
        kb2 = a_bit

        send_a = (1 - ka) * sub
        keep_a = ka * sub
        send_b = half + (1 - ka2) * sub
        keep_b = half + ka2 * sub

        qsend_a = keep_a + (1 - kb) * qtr
        qkeep_a = keep_a + kb * qtr
        qsend_b = keep_b + (1 - kb2) * qtr
        qkeep_b = keep_b + kb2 * qtr

        p1a_rows = [send_a + (1 - kb) * qtr, send_a + kb * qtr]
        p1b_rows = [send_b + kb2 * qtr, send_b + (1 - kb2) * qtr]

        def rc(src, dst, si, hf, dev):
            return pltpu.make_async_remote_copy(
                src_ref=src, dst_ref=dst,
                send_sem=send_sems.at[si, hf],
                recv_sem=recv_sems.at[si, hf],
                device_id=(dev,), device_id_type=pl.DeviceIdType.MESH,
            )

        def mm(rows):
            out_ref[pl.ds(rows, qtr), :] = jnp.dot(
                a_ref[pl.ds(rows, qtr), :], b_ref[:, :],
                preferred_element_type=jnp.float32)

        def acc(rows, buf):
            out_ref[pl.ds(rows, qtr), :] = (
                out_ref[pl.ds(rows, qtr), :] + buf[:, :]
            )

        barrier_sem = pltpu.get_barrier_semaphore()
        for nbr in [pa, pb]:
            pl.semaphore_signal(
                barrier_sem, inc=1,
                device_id=(nbr,), device_id_type=pl.DeviceIdType.MESH,
            )
        pl.semaphore_wait(barrier_sem, 2)

        mm(p1a_rows[0])
        p1a1 = rc(out_ref.at[pl.ds(p1a_rows[0], qtr), :], p1_buf.at[0, 0],
                  0, 0, pa)
        p1a1.start()
        mm(p1b_rows[0])
        p1b1 = rc(out_ref.at[pl.ds(p1b_rows[0], qtr), :], p1_buf.at[1, 0],
                  0, 1, pb)
        p1b1.start()
        mm(p1a_rows[1])
        p1a2 = rc(out_ref.at[pl.ds(p1a_rows[1], qtr), :], p1_buf.at[0, 1],
                  1, 0, pa)
        p1a2.start()
        mm(p1b_rows[1])
        p1b2 = rc(out_ref.at[pl.ds(p1b_rows[1], qtr), :], p1_buf.at[1, 1],
                  1, 1, pb)
        p1b2.start()

        mm(qsend_a)
        mm(qsend_b)
        mm(qkeep_a)
        mm(qkeep_b)

        p1a1.wait_recv()
        acc(qsend_a, p1_buf[0, 0])
        p1b1.wait_recv()
        acc(qsend_b, p1_buf[1, 0])
        p2a = rc(out_ref.at[pl.ds(qsend_a, qtr), :], p2_buf.at[0], 2, 0, pb)
        p2b = rc(out_ref.at[pl.ds(qsend_b, qtr), :], p2_buf.at[1], 2, 1, pa)
        p2a.start()
        p2b.start()
        p1a2.wait_recv()
        acc(qkeep_a, p1_buf[0, 1])
        p1b2.wait_recv()
        acc(qkeep_b, p1_buf[1, 1])

        p2a.wait_recv()
        acc(qkeep_a, p2_buf[0])
        p3a = rc(out_ref.at[pl.ds(qkeep_a, qtr), :],
                 out_ref.at[pl.ds(qkeep_a, qtr), :], 3, 0, pb)
        p3a.start()
        p4a1 = rc(out_ref.at[pl.ds(qkeep_a, qtr), :],
                  out_ref.at[pl.ds(qkeep_a, qtr), :], 4, 0, pa)
        p4a1.start()
        p2b.wait_recv()
        acc(qkeep_b, p2_buf[1])
        p3b = rc(out_ref.at[pl.ds(qkeep_b, qtr), :],
                 out_ref.at[pl.ds(qkeep_b, qtr), :], 3, 1, pa)
        p3b.start()
        p4b1 = rc(out_ref.at[pl.ds(qkeep_b, qtr), :],
                  out_ref.at[pl.ds(qkeep_b, qtr), :], 4, 1, pb)
        p4b1.start()

        p3a.wait_recv()
        p4a2 = rc(out_ref.at[pl.ds(qsend_a, qtr), :],
                  out_ref.at[pl.ds(qsend_a, qtr), :], 5, 0, pa)
        p4a2.start()
        p3b.wait_recv()
        p4b2 = rc(out_ref.at[pl.ds(qsend_b, qtr), :],
                  out_ref.at[pl.ds(qsend_b, qtr), :], 5, 1, pb)
        p4b2.start()

        p4a1.wait_recv()
        p4a2.wait_recv()
        p4b1.wait_recv()
        p4b2.wait_recv()

        for d in [p1a1, p1b1, p1a2, p1b2, p2a, p2b,
                  p3a, p3b, p4a1, p4b1, p4a2, p4b2]:
            d.wait_send()

    return pl.pallas_call(
        body,
        out_shape=jax.ShapeDtypeStruct((m, n), jnp.float32),
        in_specs=[
            pl.BlockSpec(memory_space=pltpu.VMEM),
            pl.BlockSpec(memory_space=pltpu.VMEM),
        ],
        out_specs=pl.BlockSpec(memory_space=pltpu.VMEM),
        scratch_shapes=[
            pltpu.VMEM((2, 2, qtr, n), jnp.float32),
            pltpu.VMEM((2, qtr, n), jnp.float32),
            pltpu.SemaphoreType.DMA((6, 2)),
            pltpu.SemaphoreType.DMA((6, 2)),
        ],
        compiler_params=pltpu.CompilerParams(collective_id=0),
    )(A, B)


# device time: 32199 ns/iter; 4.6476x vs baseline; 1.4890x over previous
import jax
import jax.numpy as jnp
from jax import lax
from jax.experimental import pallas as pl
from jax.experimental.pallas import tpu as pltpu

N_DEV = 4


def kernel(A, B):
    m, k = A.shape
    _, n = B.shape
    half = m // 2
    sub = m // 4
    qtr = m // 8

    f32 = jnp.float32
    bf16 = jnp.bfloat16

    def body(a_ref, b_ref, out_ref,
             s1, s2, sk, r1, r2, r3, r4, send_sems, recv_sems):
        my = lax.axis_index("i")
        a_bit = my % 2
        b_bit = my // 2
        pa = my + 1 - 2 * a_bit
        pb = 3 - my

        ka = (a_bit + b_bit) % 2
        kb = b_bit
        ka2 = b_bit
        kb2 = a_bit

        send_a = (1 - ka) * sub
        keep_a = ka * sub
        send_b = half + (1 - ka2) * sub
        keep_b = half + ka2 * sub

        qsend_a = keep_a + (1 - kb) * qtr
        qkeep_a = keep_a + kb * qtr
        qsend_b = keep_b + (1 - kb2) * qtr
        qkeep_b = keep_b + kb2 * qtr

        def rc(src, dst, si, hf, dev):
            return pltpu.make_async_remote_copy(
                src_ref=src, dst_ref=dst,
                send_sem=send_sems.at[si, hf],
                recv_sem=recv_sems.at[si, hf],
                device_id=(dev,), device_id_type=pl.DeviceIdType.MESH,
            )

        def up(x):
            return x.astype(f32)

        barrier_sem = pltpu.get_barrier_semaphore()
        for nbr in [pa, pb]:
            pl.semaphore_signal(
                barrier_sem, inc=1,
                device_id=(nbr,), device_id_type=pl.DeviceIdType.MESH,
            )
        pl.semaphore_wait(barrier_sem, 2)

        s1[0, :, :] = jnp.dot(a_ref[pl.ds(send_a, sub), :], b_ref[:, :],
                              preferred_element_type=f32).astype(bf16)
        p1a = rc(s1.at[0], r1.at[0], 0, 0, pa)
        p1a.start()
        s1[1, :, :] = jnp.dot(a_ref[pl.ds(send_b, sub), :], b_ref[:, :],
                              preferred_element_type=f32).astype(bf16)
        p1b = rc(s1.at[1], r1.at[1], 0, 1, pb)
        p1b.start()

        out_ref[pl.ds(keep_a, sub), :] = jnp.dot(
            a_ref[pl.ds(keep_a, sub), :], b_ref[:, :],
            preferred_element_type=f32)
        out_ref[pl.ds(keep_b, sub), :] = jnp.dot(
            a_ref[pl.ds(keep_b, sub), :], b_ref[:, :],
            preferred_element_type=f32)

        p1a.wait_recv()
        s2[0, :, :] = (
            out_ref[pl.ds(qsend_a, qtr), :]
            + up(r1[0, pl.ds((1 - kb) * qtr, qtr), :])
        ).astype(bf16)
        p2a = rc(s2.at[0], r2.at[0], 1, 0, pb)
        p2a.start()
        out_ref[pl.ds(qkeep_a, qtr), :] = (
            out_ref[pl.ds(qkeep_a, qtr), :]
            + up(r1[0, pl.ds(kb * qtr, qtr), :])
        )
        p1b.wait_recv()
        s2[1, :, :] = (
            out_ref[pl.ds(qsend_b, qtr), :]
            + up(r1[1, pl.ds((1 - kb2) * qtr, qtr), :])
        ).astype(bf16)
        p2b = rc(s2.at[1], r2.at[1], 1, 1, pa)
        p2b.start()
        out_ref[pl.ds(qkeep_b, qtr), :] = (
            out_ref[pl.ds(qkeep_b, qtr), :]
            + up(r1[1, pl.ds(kb2 * qtr, qtr), :])
        )

        p2a.wait_recv()
        out_ref[pl.ds(qkeep_a, qtr), :] = (
            out_ref[pl.ds(qkeep_a, qtr), :] + up(r2[0, :, :])
        )
        sk[0, :, :] = out_ref[pl.ds(qkeep_a, qtr), :].astype(bf16)
        p3a = rc(sk.at[0], r3.at[0], 2, 0, pb)
        p3a.start()
        p4a1 = rc(sk.at[0], r4.at[0, 0], 3, 0, pa)
        p4a1.start()
        p2b.wait_recv()
        out_ref[pl.ds(qkeep_b, qtr), :] = (
            out_ref[pl.ds(qkeep_b, qtr), :] + up(r2[1, :, :])
        )
        sk[1, :, :] = out_ref[pl.ds(qkeep_b, qtr), :].astype(bf16)
        p3b = rc(sk.at[1], r3.at[1], 2, 1, pa)
        p3b.start()
        p4b1 = rc(sk.at[1], r4.at[1, 0], 3, 1, pb)
        p4b1.start()

        p3a.wait_recv()
        p4a2 = rc(r3.at[0], r4.at[0, 1], 4, 0, pa)
        p4a2.start()
        out_ref[pl.ds(qsend_a, qtr), :] = up(r3[0, :, :])
        p3b.wait_recv()
        p4b2 = rc(r3.at[1], r4.at[1, 1], 4, 1, pb)
        p4b2.start()
        out_ref[pl.ds(qsend_b, qtr), :] = up(r3[1, :, :])

        p4a1.wait_recv()
        out_ref[pl.ds(send_a + kb * qtr, qtr), :] = up(r4[0, 0, :, :])
        p4b1.wait_recv()
        out_ref[pl.ds(send_b + (1 - kb2) * qtr, qtr), :] = up(r4[1, 0, :, :])
        p4a2.wait_recv()
        out_ref[pl.ds(send_a + (1 - kb) * qtr, qtr), :] = up(r4[0, 1, :, :])
        p4b2.wait_recv()
        out_ref[pl.ds(send_b + kb2 * qtr, qtr), :] = up(r4[1, 1, :, :])

        for d in [p1a, p1b, p2a, p2b, p3a, p3b, p4a1, p4b1, p4a2, p4b2]:
            d.wait_send()

    return pl.pallas_call(
        body,
        out_shape=jax.ShapeDtypeStruct((m, n), f32),
        in_specs=[
            pl.BlockSpec(memory_space=pltpu.VMEM),
            pl.BlockSpec(memory_space=pltpu.VMEM),
        ],
        out_specs=pl.BlockSpec(memory_space=pltpu.VMEM),
        scratch_shapes=[
            pltpu.VMEM((2, sub, n), bf16),
            pltpu.VMEM((2, qtr, n), bf16),
            pltpu.VMEM((2, qtr, n), bf16),
            pltpu.VMEM((2, sub, n), bf16),
            pltpu.VMEM((2, qtr, n), bf16),
            pltpu.VMEM((2, qtr, n), bf16),
            pltpu.VMEM((2, 2, qtr, n), bf16),
            pltpu.SemaphoreType.DMA((5, 2)),
            pltpu.SemaphoreType.DMA((5, 2)),
        ],
        compiler_params=pltpu.CompilerParams(collective_id=0),
    )(A, B)


# device time: 31165 ns/iter; 4.8018x vs baseline; 1.0332x over previous
import jax
import jax.numpy as jnp
from jax import lax
from jax.experimental import pallas as pl
from jax.experimental.pallas import tpu as pltpu

N_DEV = 4


def kernel(A, B):
    m, k = A.shape
    _, n = B.shape
    half = m // 2
    sub = m // 4
    qtr = m // 8

    f32 = jnp.float32
    bf16 = jnp.bfloat16

    def body(a_ref, b_ref, out_ref,
             s1, s2, sk, r1, r2, r3, r4, send_sems, recv_sems):
        my = lax.axis_index("i")
        a_bit = my % 2
        b_bit = my // 2
        pa = my + 1 - 2 * a_bit
        pb = 3 - my

        ka = (a_bit + b_bit) % 2
        kb = b_bit
        ka2 = b_bit
        kb2 = a_bit

        send_a = (1 - ka) * sub
        keep_a = ka * sub
        send_b = half + (1 - ka2) * sub
        keep_b = half + ka2 * sub

        qsend_a = keep_a + (1 - kb) * qtr
        qkeep_a = keep_a + kb * qtr
        qsend_b = keep_b + (1 - kb2) * qtr
        qkeep_b = keep_b + kb2 * qtr

        p1a_rows = [send_a + (1 - kb) * qtr, send_a + kb * qtr]
        p1b_rows = [send_b + kb2 * qtr, send_b + (1 - kb2) * qtr]

        def rc(src, dst, si, hf, dev):
            return pltpu.make_async_remote_copy(
                src_ref=src, dst_ref=dst,
                send_sem=send_sems.at[si, hf],
                recv_sem=recv_sems.at[si, hf],
                device_id=(dev,), device_id_type=pl.DeviceIdType.MESH,
            )

        def up(x):
            return x.astype(f32)

        barrier_sem = pltpu.get_barrier_semaphore()
        for nbr in [pa, pb]:
            pl.semaphore_signal(
                barrier_sem, inc=1,
                device_id=(nbr,), device_id_type=pl.DeviceIdType.MESH,
            )
        pl.semaphore_wait(barrier_sem, 2)

        def mm_bf16(rows, dst):
            dst[:, :] = jnp.dot(a_ref[pl.ds(rows, qtr), :], b_ref[:, :],
                                preferred_element_type=f32).astype(bf16)

        mm_bf16(p1a_rows[0], s1.at[0, 0])
        p1a1 = rc(s1.at[0, 0], r1.at[0, 0], 0, 0, pa)
        p1a1.start()
        mm_bf16(p1b_rows[0], s1.at[1, 0])
        p1b1 = rc(s1.at[1, 0], r1.at[1, 0], 0, 1, pb)
        p1b1.start()
        mm_bf16(p1a_rows[1], s1.at[0, 1])
        p1a2 = rc(s1.at[0, 1], r1.at[0, 1], 1, 0, pa)
        p1a2.start()
        mm_bf16(p1b_rows[1], s1.at[1, 1])
        p1b2 = rc(s1.at[1, 1], r1.at[1, 1], 1, 1, pb)
        p1b2.start()

        def mm_f32(rows):
            out_ref[pl.ds(rows, qtr), :] = jnp.dot(
                a_ref[pl.ds(rows, qtr), :], b_ref[:, :],
                preferred_element_type=f32)

        mm_f32(qsend_a)
        mm_f32(qsend_b)
        mm_f32(qkeep_a)
        mm_f32(qkeep_b)

        p1a1.wait_recv()
        s2[0, :, :] = (
            out_ref[pl.ds(qsend_a, qtr), :] + up(r1[0, 0, :, :])
        ).astype(bf16)
        p2a = rc(s2.at[0], r2.at[0], 2, 0, pb)
        p2a.start()
        p1b1.wait_recv()
        s2[1, :, :] = (
            out_ref[pl.ds(qsend_b, qtr), :] + up(r1[1, 0, :, :])
        ).astype(bf16)
        p2b = rc(s2.at[1], r2.at[1], 2, 1, pa)
        p2b.start()
        p1a2.wait_recv()
        out_ref[pl.ds(qkeep_a, qtr), :] = (
            out_ref[pl.ds(qkeep_a, qtr), :] + up(r1[0, 1, :, :])
        )
        p1b2.wait_recv()
        out_ref[pl.ds(qkeep_b, qtr), :] = (
            out_ref[pl.ds(qkeep_b, qtr), :] + up(r1[1, 1, :, :])
        )

        p2a.wait_recv()
        out_ref[pl.ds(qkeep_a, qtr), :] = (
            out_ref[pl.ds(qkeep_a, qtr), :] + up(r2[0, :, :])
        )
        sk[0, :, :] = out_ref[pl.ds(qkeep_a, qtr), :].astype(bf16)
        p3a = rc(sk.at[0], r3.at[0], 3, 0, pb)
        p3a.start()
        p4a1 = rc(sk.at[0], r4.at[0, 0], 4, 0, pa)
        p4a1.start()
        p2b.wait_recv()
        out_ref[pl.ds(qkeep_b, qtr), :] = (
            out_ref[pl.ds(qkeep_b, qtr), :] + up(r2[1, :, :])
        )
        sk[1, :, :] = out_ref[pl.ds(qkeep_b, qtr), :].astype(bf16)
        p3b = rc(sk.at[1], r3.at[1], 3, 1, pa)
        p3b.start()
        p4b1 = rc(sk.at[1], r4.at[1, 0], 4, 1, pb)
        p4b1.start()

        p3a.wait_recv()
        p4a2 = rc(r3.at[0], r4.at[0, 1], 5, 0, pa)
        p4a2.start()
        out_ref[pl.ds(qsend_a, qtr), :] = up(r3[0, :, :])
        p3b.wait_recv()
        p4b2 = rc(r3.at[1], r4.at[1, 1], 5, 1, pb)
        p4b2.start()
        out_ref[pl.ds(qsend_b, qtr), :] = up(r3[1, :, :])

        p4a1.wait_recv()
        out_ref[pl.ds(send_a + kb * qtr, qtr), :] = up(r4[0, 0, :, :])
        p4b1.wait_recv()
        out_ref[pl.ds(send_b + (1 - kb2) * qtr, qtr), :] = up(r4[1, 0, :, :])
        p4a2.wait_recv()
        out_ref[pl.ds(send_a + (1 - kb) * qtr, qtr), :] = up(r4[0, 1, :, :])
        p4b2.wait_recv()
        out_ref[pl.ds(send_b + kb2 * qtr, qtr), :] = up(r4[1, 1, :, :])

        for d in [p1a1, p1b1, p1a2, p1b2, p2a, p2b, p3a, p3b,
                  p4a1, p4b1, p4a2, p4b2]:
            d.wait_send()

    return pl.pallas_call(
        body,
        out_shape=jax.ShapeDtypeStruct((m, n), f32),
        in_specs=[
            pl.BlockSpec(memory_space=pltpu.VMEM),
            pl.BlockSpec(memory_space=pltpu.VMEM),
        ],
        out_specs=pl.BlockSpec(memory_space=pltpu.VMEM),
        scratch_shapes=[
            pltpu.VMEM((2, 2, qtr, n), bf16),
            pltpu.VMEM((2, qtr, n), bf16),
            pltpu.VMEM((2, qtr, n), bf16),
            pltpu.VMEM((2, 2, qtr, n), bf16),
            pltpu.VMEM((2, qtr, n), bf16),
            pltpu.VMEM((2, qtr, n), bf16),
            pltpu.VMEM((2, 2, qtr, n), bf16),
            pltpu.SemaphoreType.DMA((6, 2)),
            pltpu.SemaphoreType.DMA((6, 2)),
        ],
        compiler_params=pltpu.CompilerParams(collective_id=0),
    )(A, B)


# device time: 6622 ns/iter; 22.5986x vs baseline; 4.7063x over previous
import jax
import jax.numpy as jnp
from jax import lax
from jax.experimental import pallas as pl
from jax.experimental.pallas import tpu as pltpu

N_DEV = 4


def kernel(A, B):
    m, k = A.shape
    _, n = B.shape
    half = m // 2
    sub = m // 4
    qtr = m // 8

    f32 = jnp.float32
    bf16 = jnp.bfloat16

    def body(a_ref, b_ref, out_ref,
             s1, s2, s3, r1, r2, r3, send_sems, recv_sems):
        my = lax.axis_index("i")
        a_bit = my % 2
        b_bit = my // 2
        pa = my + 1 - 2 * a_bit
        pb = 3 - my

        ka = (a_bit + b_bit) % 2
        ka2 = b_bit

        send_a = (1 - ka) * sub
        keep_a = ka * sub
        send_b = half + (1 - ka2) * sub
        keep_b = half + ka2 * sub

        def rc(src, dst, si, hf, dev):
            return pltpu.make_async_remote_copy(
                src_ref=src, dst_ref=dst,
                send_sem=send_sems.at[si, hf],
                recv_sem=recv_sems.at[si, hf],
                device_id=(dev,), device_id_type=pl.DeviceIdType.MESH,
            )

        def up(x):
            return x.astype(f32)

        barrier_sem = pltpu.get_barrier_semaphore()
        for nbr in [pa, pb]:
            pl.semaphore_signal(
                barrier_sem, inc=1,
                device_id=(nbr,), device_id_type=pl.DeviceIdType.MESH,
            )
        pl.semaphore_wait(barrier_sem, 2)

        def mm_bf16(rows, dst):
            dst[:, :] = jnp.dot(a_ref[pl.ds(rows, qtr), :], b_ref[:, :],
                                preferred_element_type=f32).astype(bf16)

        def mm_f32(rows):
            out_ref[pl.ds(rows, qtr), :] = jnp.dot(
                a_ref[pl.ds(rows, qtr), :], b_ref[:, :],
                preferred_element_type=f32)

        p1 = [[None, None], [None, None]]
        for p in range(2):
            mm_bf16(send_a + p * qtr, s1.at[0, p])
            p1[0][p] = rc(s1.at[0, p], r1.at[0, p], 0 + p, 0, pa)
            p1[0][p].start()
            mm_bf16(send_b + p * qtr, s1.at[1, p])
            p1[1][p] = rc(s1.at[1, p], r1.at[1, p], 0 + p, 1, pb)
            p1[1][p].start()

        mm_f32(keep_a)
        mm_f32(keep_b)
        mm_f32(keep_a + qtr)
        mm_f32(keep_b + qtr)

        p2 = [[None, None], [None, None]]
        keeps = [keep_a, keep_b]
        partner2 = [pb, pa]
        for p in range(2):
            for hf in range(2):
                rows = keeps[hf] + p * qtr
                p1[hf][p].wait_recv()
                s2[hf, p, :, :] = (
                    out_ref[pl.ds(rows, qtr), :] + up(r1[hf, p, :, :])
                ).astype(bf16)
                p2[hf][p] = rc(s2.at[hf, p], r2.at[hf, p],
                               2 + p, hf, partner2[hf])
                p2[hf][p].start()
                out_ref[pl.ds(rows, qtr), :] = (
                    out_ref[pl.ds(rows, qtr), :] + up(r1[hf, p, :, :])
                )

        p3 = [[None, None], [None, None]]
        partner3 = [pa, pb]
        for p in range(2):
            for hf in range(2):
                rows = keeps[hf] + p * qtr
                p2[hf][p].wait_recv()
                out_ref[pl.ds(rows, qtr), :] = (
                    out_ref[pl.ds(rows, qtr), :] + up(r2[hf, p, :, :])
                )
                s3[hf, p, :, :] = out_ref[pl.ds(rows, qtr), :].astype(bf16)
                p3[hf][p] = rc(s3.at[hf, p], r3.at[hf, p],
                               4 + p, hf, partner3[hf])
                p3[hf][p].start()

        sends = [send_a, send_b]
        for p in range(2):
            for hf in range(2):
                p3[hf][p].wait_recv()
                out_ref[pl.ds(sends[hf] + p * qtr, qtr), :] = up(
                    r3[hf, p, :, :]
                )

        for group in [p1, p2, p3]:
            for hf in range(2):
                for p in range(2):
                    group[hf][p].wait_send()

    return pl.pallas_call(
        body,
        out_shape=jax.ShapeDtypeStruct((m, n), f32),
        in_specs=[
            pl.BlockSpec(memory_space=pltpu.VMEM),
            pl.BlockSpec(memory_space=pltpu.VMEM),
        ],
        out_specs=pl.BlockSpec(memory_space=pltpu.VMEM),
        scratch_shapes=[
            pltpu.VMEM((2, 2, qtr, n), bf16),
            pltpu.VMEM((2, 2, qtr, n), bf16),
            pltpu.VMEM((2, 2, qtr, n), bf16),
            pltpu.VMEM((2, 2, qtr, n), bf16),
            pltpu.VMEM((2, 2, qtr, n), bf16),
            pltpu.VMEM((2, 2, qtr, n), bf16),
            pltpu.SemaphoreType.DMA((6, 2)),
            pltpu.SemaphoreType.DMA((6, 2)),
        ],
        compiler_params=pltpu.CompilerParams(collective_id=0),
    )(A, B)
